# baseline (speedup 1.0000x reference)
import jax
import jax.numpy as jnp
from jax import lax
from jax.experimental import pallas as pl
from jax.experimental.pallas import tpu as pltpu

_ROWS = 16384
_COLS = 200
_RB = 8
_GRID = _COLS // _RB
_NBUF = 4


def _tc_body(w_ref, x_ref, out_ref, acc_ref, sem):
    i = pl.program_id(0)
    b = lax.rem(i, _NBUF)

    def dma(step, buf, r):
        return pltpu.make_async_copy(
            acc_ref.at[buf, r], out_ref.at[step * _RB + r, 0], sem)

    @pl.when(i >= _NBUF - 1)
    def _():
        j = i - (_NBUF - 1)
        for r in range(_RB):
            dma(j, lax.rem(j, _NBUF), r).wait()

    xb = x_ref[...]
    w0 = w_ref[0, 0]
    w1 = w_ref[0, 1]
    w2 = w_ref[0, 2]
    w3 = w_ref[0, 3]
    lo = jnp.where(xb == 1, w1, w0)
    hi = jnp.where(xb == 3, w3, w2)
    acc_ref[b] = jnp.where(xb >= 2, hi, lo)
    for r in range(_RB):
        dma(i, b, r).start()

    @pl.when(i == _GRID - 1)
    def _():
        for j in range(_GRID - (_NBUF - 1), _GRID):
            for r in range(_RB):
                dma(j, j % _NBUF, r).wait()


@jax.jit
def kernel(x, weight):
    w_row = weight.reshape(1, 4).astype(jnp.float32)
    xt = x.T
    out_lin = pl.pallas_call(
        _tc_body,
        grid=(_GRID,),
        in_specs=[
            pl.BlockSpec(memory_space=pltpu.SMEM),
            pl.BlockSpec((_RB, _ROWS), lambda i: (i, 0)),
        ],
        out_specs=pl.BlockSpec(memory_space=pl.ANY),
        out_shape=jax.ShapeDtypeStruct((_COLS, 1, _ROWS), jnp.float32),
        scratch_shapes=[
            pltpu.VMEM((_NBUF, _RB, _ROWS), jnp.float32),
            pltpu.SemaphoreType.DMA,
        ],
    )(w_row, xt)
    return jnp.transpose(out_lin, (2, 0, 1))
